# baseline (speedup 1.0000x reference)
import functools

import jax
import jax.numpy as jnp
from jax import lax
from jax.experimental import pallas as pl
from jax.experimental.pallas import tpu as pltpu
from jax.experimental.pallas import tpu_sc as plsc

_N = 10000
_E = 160000
_F = 128
_NB = 8
_SHD = 9
_NSPEC = 10
_INV_SQRT_AVG = 0.25

_NC = 2
_NS = 16
_NW = _NC * _NS
_EPW = _E // _NW
_CH = 40
_NCHUNK = _EPW // _CH
_NPAD = 10240
_NPT = _NPAD // _NS

def _sc_gather_body(nf_hbm, snd3_hbm, out_hbm, idx_v, rows_v, sem):
    c = lax.axis_index("c")
    s = lax.axis_index("s")
    wid = c * _NS + s
    base0 = wid * _EPW
    pltpu.sync_copy(snd3_hbm.at[wid], idx_v)

    def body(i, _):
        pltpu.async_copy(nf_hbm.at[idx_v.at[i]], rows_v, sem).wait()
        pltpu.sync_copy(rows_v, out_hbm.at[pl.ds(base0 + i * _CH, _CH), :])
        return ()

    lax.fori_loop(0, _NCHUNK, body, (), unroll=False)


def _sc_scatter_body(m_hbm, rcv3_hbm, zeros_hbm, out_hbm, acc_sh, idx_v,
                     rows_v, sem):
    c = lax.axis_index("c")
    s = lax.axis_index("s")
    wid = c * _NS + s
    base0 = wid * _EPW
    nbase = s * _NPT
    pltpu.sync_copy(zeros_hbm, acc_sh.at[pl.ds(nbase, _NPT), :])
    pltpu.sync_copy(rcv3_hbm.at[wid], idx_v)
    plsc.subcore_barrier()

    def body(i, _):
        pltpu.sync_copy(m_hbm.at[pl.ds(base0 + i * _CH, _CH), :], rows_v)
        pltpu.sync_copy(rows_v, acc_sh.at[idx_v.at[i]], add=True)
        return ()

    lax.fori_loop(0, _NCHUNK, body, (), unroll=False)
    plsc.subcore_barrier()
    pltpu.sync_copy(acc_sh.at[pl.ds(nbase, _NPT), :],
                    out_hbm.at[c, pl.ds(nbase, _NPT), :])


@functools.lru_cache(maxsize=None)
def _sc_impls():
    mesh = plsc.VectorSubcoreMesh(core_axis_name="c", subcore_axis_name="s",
                                  num_cores=_NC, num_subcores=_NS)
    gather = pl.kernel(
        _sc_gather_body,
        out_type=jax.ShapeDtypeStruct((_E, _F), jnp.float32),
        mesh=mesh,
        scratch_types=[
            pltpu.VMEM((_NCHUNK, _CH), jnp.int32),
            pltpu.VMEM((_CH, _F), jnp.float32),
            pltpu.SemaphoreType.DMA,
        ],
    )
    scatter = pl.kernel(
        _sc_scatter_body,
        out_type=jax.ShapeDtypeStruct((_NC, _NPAD, _F), jnp.float32),
        mesh=mesh,
        scratch_types=[
            pltpu.VMEM_SHARED((_NPAD, _F), jnp.float32),
            pltpu.VMEM((_NCHUNK, _CH), jnp.int32),
            pltpu.VMEM((_CH, _F), jnp.float32),
            pltpu.SemaphoreType.DMA,
        ],
    )
    return gather, scatter


_TE = 1280


def _tc_edge_body(vec_ref, rad_ref, h_ref, wr1_ref, br1_ref, wr2_ref, br2_ref,
                  wlin_ref, m_ref):
    v = vec_ref[...]
    r = jnp.sqrt(jnp.sum(v * v, axis=1, keepdims=True)) + 1e-8
    u = v / r
    x, y, z = u[:, 0:1], u[:, 1:2], u[:, 2:3]
    rh = jnp.dot(rad_ref[...], wr1_ref[...],
                 preferred_element_type=jnp.float32) + br1_ref[...]
    rh = rh * (1.0 / (1.0 + jnp.exp(-rh)))
    rw = jnp.dot(rh, wr2_ref[...],
                 preferred_element_type=jnp.float32) + br2_ref[...]
    h = h_ref[...]
    ys = (jnp.ones_like(x), x, y, z,
          x * y, y * z, 3.0 * z * z - 1.0, x * z, x * x - y * y)
    cols = []
    for lm in range(_SHD):
        coef = ys[lm] * rw[:, lm:lm + 1]
        cols.append((coef * h).astype(jnp.bfloat16))
    xedge = jnp.concatenate(cols, axis=1)
    m_ref[...] = jnp.dot(xedge, wlin_ref[...],
                         preferred_element_type=jnp.float32)


def _tc_edge(vectors, radial, h_send, wr1, br1, wr2, br2, wlin):
    grid = (_E // _TE,)
    return pl.pallas_call(
        _tc_edge_body,
        grid=grid,
        in_specs=[
            pl.BlockSpec((_TE, 3), lambda i: (i, 0)),
            pl.BlockSpec((_TE, _NB), lambda i: (i, 0)),
            pl.BlockSpec((_TE, _F), lambda i: (i, 0)),
            pl.BlockSpec((_NB, 64), lambda i: (0, 0)),
            pl.BlockSpec((1, 64), lambda i: (0, 0)),
            pl.BlockSpec((64, _SHD), lambda i: (0, 0)),
            pl.BlockSpec((1, _SHD), lambda i: (0, 0)),
            pl.BlockSpec((_SHD * _F, _F), lambda i: (0, 0)),
        ],
        out_specs=pl.BlockSpec((_TE, _F), lambda i: (i, 0)),
        out_shape=jax.ShapeDtypeStruct((_E, _F), jnp.float32),
    )(vectors, radial, h_send, wr1, br1, wr2, br2, wlin)


_TN = 1000


def _tc_node_body(p0_ref, p1_ref, nf_ref, spec_ref, wskip_ref, wprod_ref,
                  wpl_ref, wread_ref, out1_ref, feats_ref):
    f = (p0_ref[...] + p1_ref[...]) * _INV_SQRT_AVG
    spec = spec_ref[...]
    nf = nf_ref[...]
    parts = [jnp.where(spec == s, nf, 0.0) for s in range(_NSPEC)]
    xcat = jnp.concatenate(parts, axis=1)
    sc = jnp.dot(xcat, wskip_ref[...], preferred_element_type=jnp.float32)
    iota = lax.broadcasted_iota(jnp.int32, (1, _NSPEC), 1)
    onehot = (spec == iota).astype(jnp.float32)
    w = jnp.dot(onehot, wprod_ref[...], preferred_element_type=jnp.float32)
    w0, w1, w2 = w[:, :_F], w[:, _F:2 * _F], w[:, 2 * _F:3 * _F]
    pb = (w0 + w1 * f + w2 * (f * f)) * f
    feats = jnp.dot(pb, wpl_ref[...], preferred_element_type=jnp.float32) + sc
    feats_ref[...] = feats
    out1_ref[...] = jnp.dot(feats, wread_ref[...],
                            preferred_element_type=jnp.float32)


def _tc_node(p0, p1, node_feats, spec2, wskip_flat, wprod2, wpl, wread):
    grid = (_N // _TN,)
    return pl.pallas_call(
        _tc_node_body,
        grid=grid,
        in_specs=[
            pl.BlockSpec((_TN, _F), lambda i: (i, 0)),
            pl.BlockSpec((_TN, _F), lambda i: (i, 0)),
            pl.BlockSpec((_TN, _F), lambda i: (i, 0)),
            pl.BlockSpec((_TN, 1), lambda i: (i, 0)),
            pl.BlockSpec((_NSPEC * _F, _F), lambda i: (0, 0)),
            pl.BlockSpec((_NSPEC, 3 * _F), lambda i: (0, 0)),
            pl.BlockSpec((_F, _F), lambda i: (0, 0)),
            pl.BlockSpec((_F, 1), lambda i: (0, 0)),
        ],
        out_specs=[
            pl.BlockSpec((_TN, 1), lambda i: (i, 0)),
            pl.BlockSpec((_TN, _F), lambda i: (i, 0)),
        ],
        out_shape=[
            jax.ShapeDtypeStruct((_N, 1), jnp.float32),
            jax.ShapeDtypeStruct((_N, _F), jnp.float32),
        ],
    )(p0, p1, node_feats, spec2, wskip_flat, wprod2, wpl, wread)


def kernel(vectors, node_feats, node_specie, radial_embedding, senders,
           receivers, W_skip, Wr1, br1, Wr2, br2, W_lin, w_prod, W_prodlin,
           W_read):
    snd3 = senders.astype(jnp.int32).reshape(_NW, _NCHUNK, _CH)
    rcv3 = receivers.astype(jnp.int32).reshape(_NW, _NCHUNK, _CH)
    wlin_bf = W_lin.astype(jnp.bfloat16)
    zeros_tile = jnp.zeros((_NPT, _F), jnp.float32)

    sc_gather, sc_scatter = _sc_impls()
    h_send = sc_gather(node_feats, snd3)
    m = _tc_edge(vectors, radial_embedding, h_send, Wr1,
                 br1.reshape(1, 64), Wr2, br2.reshape(1, _SHD), wlin_bf)
    partials = sc_scatter(m, rcv3, zeros_tile)
    p0 = partials[0, :_N]
    p1 = partials[1, :_N]

    spec2 = node_specie.astype(jnp.int32).reshape(_N, 1)
    wskip_flat = W_skip.reshape(_NSPEC * _F, _F)
    wprod2 = w_prod.reshape(_NSPEC, 3 * _F)
    node_outputs, feats = _tc_node(p0, p1, node_feats, spec2,
                                   wskip_flat, wprod2, W_prodlin, W_read)
    return node_outputs, feats


# speedup vs baseline: 1.4341x; 1.4341x over previous
import functools

import jax
import jax.numpy as jnp
from jax import lax
from jax.experimental import pallas as pl
from jax.experimental.pallas import tpu as pltpu
from jax.experimental.pallas import tpu_sc as plsc

_N = 10000
_E = 160000
_F = 128
_NB = 8
_SHD = 9
_NSPEC = 10
_INV_SQRT_AVG = 0.25

_NC = 2
_NS = 16
_NW = _NC * _NS
_EPW = _E // _NW
_CH = 40
_NCHUNK = _EPW // _CH
_NPAD = 10240
_NPT = _NPAD // _NS

def _sc_gather_body(nf_hbm, snd3_hbm, out_hbm, idx_v, rows_v, sem):
    c = lax.axis_index("c")
    s = lax.axis_index("s")
    wid = c * _NS + s
    base0 = wid * _EPW
    pltpu.sync_copy(snd3_hbm.at[wid], idx_v)

    def body(i, _):
        pltpu.async_copy(nf_hbm.at[idx_v.at[i]], rows_v, sem).wait()
        pltpu.sync_copy(rows_v, out_hbm.at[pl.ds(base0 + i * _CH, _CH), :])
        return ()

    lax.fori_loop(0, _NCHUNK, body, (), unroll=False)


def _sc_scatter_body(m_hbm, rcv3_hbm, zeros_hbm, out_hbm, acc_sh, idx_v,
                     rows_v, sem):
    c = lax.axis_index("c")
    s = lax.axis_index("s")
    wid = c * _NS + s
    base0 = wid * _EPW
    nbase = s * _NPT
    pltpu.sync_copy(zeros_hbm, acc_sh.at[pl.ds(nbase, _NPT), :])
    pltpu.sync_copy(rcv3_hbm.at[wid], idx_v)
    plsc.subcore_barrier()

    def body(i, _):
        pltpu.sync_copy(m_hbm.at[pl.ds(base0 + i * _CH, _CH), :], rows_v)
        pltpu.sync_copy(rows_v, acc_sh.at[idx_v.at[i]], add=True)
        return ()

    lax.fori_loop(0, _NCHUNK, body, (), unroll=False)
    plsc.subcore_barrier()
    pltpu.sync_copy(acc_sh.at[pl.ds(nbase, _NPT), :],
                    out_hbm.at[c, pl.ds(nbase, _NPT), :])


@functools.lru_cache(maxsize=None)
def _sc_impls():
    mesh = plsc.VectorSubcoreMesh(core_axis_name="c", subcore_axis_name="s",
                                  num_cores=_NC, num_subcores=_NS)
    gather = pl.kernel(
        _sc_gather_body,
        out_type=jax.ShapeDtypeStruct((_E, _F), jnp.float32),
        mesh=mesh,
        scratch_types=[
            pltpu.VMEM((_NCHUNK, _CH), jnp.int32),
            pltpu.VMEM((_CH, _F), jnp.float32),
            pltpu.SemaphoreType.DMA,
        ],
    )
    scatter = pl.kernel(
        _sc_scatter_body,
        out_type=jax.ShapeDtypeStruct((_NC, _NPAD, _F), jnp.float32),
        mesh=mesh,
        scratch_types=[
            pltpu.VMEM_SHARED((_NPAD, _F), jnp.float32),
            pltpu.VMEM((_NCHUNK, _CH), jnp.int32),
            pltpu.VMEM((_CH, _F), jnp.float32),
            pltpu.SemaphoreType.DMA,
        ],
    )
    return gather, scatter


_CHK = 3200


def _tc_coef_body(vt_ref, radt_ref, wr1t_ref, br1t_ref, wr2t_ref, br2t_ref,
                  ct_ref):
    v = vt_ref[...]
    rsq = jnp.sum(v * v, axis=0, keepdims=True)
    inv = 1.0 / (jnp.sqrt(rsq) + 1e-8)
    x = v[0:1, :] * inv
    y = v[1:2, :] * inv
    z = v[2:3, :] * inv
    rh = jnp.dot(wr1t_ref[...], radt_ref[...],
                 preferred_element_type=jnp.float32) + br1t_ref[...]
    rh = rh * (1.0 / (1.0 + jnp.exp(-rh)))
    rw = jnp.dot(wr2t_ref[...], rh,
                 preferred_element_type=jnp.float32) + br2t_ref[...]
    yt = jnp.concatenate([jnp.ones_like(x), x, y, z,
                          x * y, y * z, 3.0 * z * z - 1.0, x * z,
                          x * x - y * y], axis=0)
    ct_ref[...] = (yt * rw).astype(jnp.bfloat16)


def _tc_coef(vt, radt, wr1t, br1t, wr2t, br2t):
    grid = (_E // _CHK,)
    return pl.pallas_call(
        _tc_coef_body,
        grid=grid,
        in_specs=[
            pl.BlockSpec((3, _CHK), lambda i: (0, i)),
            pl.BlockSpec((_NB, _CHK), lambda i: (0, i)),
            pl.BlockSpec((64, _NB), lambda i: (0, 0)),
            pl.BlockSpec((64, 1), lambda i: (0, 0)),
            pl.BlockSpec((_SHD, 64), lambda i: (0, 0)),
            pl.BlockSpec((_SHD, 1), lambda i: (0, 0)),
        ],
        out_specs=pl.BlockSpec((_SHD, _CHK), lambda i: (0, i)),
        out_shape=jax.ShapeDtypeStruct((_SHD, _E), jnp.bfloat16),
    )(vt, radt, wr1t, br1t, wr2t, br2t)


_TE = 1280


def _tc_edge_body(ct_ref, h_ref, bsel_ref, wlin_ref, m_ref):
    h_bf = h_ref[...].astype(jnp.bfloat16)
    pmat = lax.dot_general(
        ct_ref[...], bsel_ref[...],
        dimension_numbers=(((0,), (0,)), ((), ())),
        preferred_element_type=jnp.float32).astype(jnp.bfloat16)
    xedge = jnp.concatenate(
        [pmat[:, lm * _F:(lm + 1) * _F] * h_bf for lm in range(_SHD)], axis=1)
    m_ref[...] = jnp.dot(xedge, wlin_ref[...],
                         preferred_element_type=jnp.float32)


def _tc_edge(ct, h_send, bsel, wlin):
    grid = (_E // _TE,)
    return pl.pallas_call(
        _tc_edge_body,
        grid=grid,
        in_specs=[
            pl.BlockSpec((_SHD, _TE), lambda i: (0, i)),
            pl.BlockSpec((_TE, _F), lambda i: (i, 0)),
            pl.BlockSpec((_SHD, _SHD * _F), lambda i: (0, 0)),
            pl.BlockSpec((_SHD * _F, _F), lambda i: (0, 0)),
        ],
        out_specs=pl.BlockSpec((_TE, _F), lambda i: (i, 0)),
        out_shape=jax.ShapeDtypeStruct((_E, _F), jnp.float32),
    )(ct, h_send, bsel, wlin)


_TN = 1000


def _tc_node_body(p0_ref, p1_ref, nf_ref, spec_ref, wskip_ref, wprod_ref,
                  wpl_ref, wread_ref, out1_ref, feats_ref):
    f = (p0_ref[...] + p1_ref[...]) * _INV_SQRT_AVG
    spec = spec_ref[...]
    nf = nf_ref[...]
    parts = [jnp.where(spec == s, nf, 0.0) for s in range(_NSPEC)]
    xcat = jnp.concatenate(parts, axis=1)
    sc = jnp.dot(xcat, wskip_ref[...], preferred_element_type=jnp.float32)
    iota = lax.broadcasted_iota(jnp.int32, (1, _NSPEC), 1)
    onehot = (spec == iota).astype(jnp.float32)
    w = jnp.dot(onehot, wprod_ref[...], preferred_element_type=jnp.float32)
    w0, w1, w2 = w[:, :_F], w[:, _F:2 * _F], w[:, 2 * _F:3 * _F]
    pb = (w0 + w1 * f + w2 * (f * f)) * f
    feats = jnp.dot(pb, wpl_ref[...], preferred_element_type=jnp.float32) + sc
    feats_ref[...] = feats
    out1_ref[...] = jnp.dot(feats, wread_ref[...],
                            preferred_element_type=jnp.float32)


def _tc_node(p0, p1, node_feats, spec2, wskip_flat, wprod2, wpl, wread):
    grid = (_N // _TN,)
    return pl.pallas_call(
        _tc_node_body,
        grid=grid,
        in_specs=[
            pl.BlockSpec((_TN, _F), lambda i: (i, 0)),
            pl.BlockSpec((_TN, _F), lambda i: (i, 0)),
            pl.BlockSpec((_TN, _F), lambda i: (i, 0)),
            pl.BlockSpec((_TN, 1), lambda i: (i, 0)),
            pl.BlockSpec((_NSPEC * _F, _F), lambda i: (0, 0)),
            pl.BlockSpec((_NSPEC, 3 * _F), lambda i: (0, 0)),
            pl.BlockSpec((_F, _F), lambda i: (0, 0)),
            pl.BlockSpec((_F, 1), lambda i: (0, 0)),
        ],
        out_specs=[
            pl.BlockSpec((_TN, 1), lambda i: (i, 0)),
            pl.BlockSpec((_TN, _F), lambda i: (i, 0)),
        ],
        out_shape=[
            jax.ShapeDtypeStruct((_N, 1), jnp.float32),
            jax.ShapeDtypeStruct((_N, _F), jnp.float32),
        ],
    )(p0, p1, node_feats, spec2, wskip_flat, wprod2, wpl, wread)


def kernel(vectors, node_feats, node_specie, radial_embedding, senders,
           receivers, W_skip, Wr1, br1, Wr2, br2, W_lin, w_prod, W_prodlin,
           W_read):
    snd3 = senders.astype(jnp.int32).reshape(_NW, _NCHUNK, _CH)
    rcv3 = receivers.astype(jnp.int32).reshape(_NW, _NCHUNK, _CH)
    wlin_bf = W_lin.astype(jnp.bfloat16)
    bsel = jnp.repeat(jnp.eye(_SHD, dtype=jnp.bfloat16), _F, axis=1)
    zeros_tile = jnp.zeros((_NPT, _F), jnp.float32)

    sc_gather, sc_scatter = _sc_impls()
    h_send = sc_gather(node_feats, snd3)
    ct = _tc_coef(vectors.T, radial_embedding.T, Wr1.T,
                  br1.reshape(64, 1), Wr2.T, br2.reshape(_SHD, 1))
    m = _tc_edge(ct, h_send, bsel, wlin_bf)
    partials = sc_scatter(m, rcv3, zeros_tile)
    p0 = partials[0, :_N]
    p1 = partials[1, :_N]

    spec2 = node_specie.astype(jnp.int32).reshape(_N, 1)
    wskip_flat = W_skip.reshape(_NSPEC * _F, _F)
    wprod2 = w_prod.reshape(_NSPEC, 3 * _F)
    node_outputs, feats = _tc_node(p0, p1, node_feats, spec2,
                                   wskip_flat, wprod2, W_prodlin, W_read)
    return node_outputs, feats


# speedup vs baseline: 1.5833x; 1.1040x over previous
import functools

import jax
import jax.numpy as jnp
from jax import lax
from jax.experimental import pallas as pl
from jax.experimental.pallas import tpu as pltpu
from jax.experimental.pallas import tpu_sc as plsc

_N = 10000
_E = 160000
_F = 128
_NB = 8
_SHD = 9
_NSPEC = 10
_INV_SQRT_AVG = 0.25

_NC = 2
_NS = 16
_NW = _NC * _NS
_EPW = _E // _NW
_CH = 40
_NCHUNK = _EPW // _CH
_NPAD = 10240
_NPT = _NPAD // _NS

def _sc_gather_body(nf_hbm, snd3_hbm, out_hbm, idx_v, rows_v, sem0, sem1):
    c = lax.axis_index("c")
    s = lax.axis_index("s")
    wid = c * _NS + s
    base0 = wid * _EPW
    pltpu.sync_copy(snd3_hbm.at[wid], idx_v)
    sems = (sem0, sem1)

    def start(i, b):
        pltpu.async_copy(nf_hbm.at[idx_v.at[i]], rows_v.at[b], sems[b])

    def finish(i, b):
        pltpu.make_async_copy(nf_hbm.at[idx_v.at[i]], rows_v.at[b],
                              sems[b]).wait()
        pltpu.sync_copy(rows_v.at[b],
                        out_hbm.at[pl.ds(base0 + i * _CH, _CH), :])

    start(0, 0)

    def pair(j, _):
        i0 = j * 2
        pltpu.make_async_copy(nf_hbm.at[idx_v.at[i0]], rows_v.at[0],
                              sem0).wait()
        start(i0 + 1, 1)
        pltpu.sync_copy(rows_v.at[0],
                        out_hbm.at[pl.ds(base0 + i0 * _CH, _CH), :])
        pltpu.make_async_copy(nf_hbm.at[idx_v.at[i0 + 1]], rows_v.at[1],
                              sem1).wait()
        start(i0 + 2, 0)
        pltpu.sync_copy(rows_v.at[1],
                        out_hbm.at[pl.ds(base0 + (i0 + 1) * _CH, _CH), :])
        return ()

    lax.fori_loop(0, _NCHUNK // 2, pair, (), unroll=False)
    finish(_NCHUNK - 1, 0)


def _sc_scatter_body(m_hbm, rcv3_hbm, zeros_hbm, out_hbm, acc_sh, idx_v,
                     rows_v, sem0, sem1):
    c = lax.axis_index("c")
    s = lax.axis_index("s")
    wid = c * _NS + s
    base0 = wid * _EPW
    nbase = s * _NPT
    pltpu.sync_copy(zeros_hbm, acc_sh.at[pl.ds(nbase, _NPT), :])
    pltpu.sync_copy(rcv3_hbm.at[wid], idx_v)
    plsc.subcore_barrier()

    sems = (sem0, sem1)

    def start(i, b):
        pltpu.async_copy(m_hbm.at[pl.ds(base0 + i * _CH, _CH), :],
                         rows_v.at[b], sems[b])

    def wait(i, b):
        pltpu.make_async_copy(m_hbm.at[pl.ds(base0 + i * _CH, _CH), :],
                              rows_v.at[b], sems[b]).wait()

    start(0, 0)

    def pair(j, _):
        i0 = j * 2
        wait(i0, 0)
        start(i0 + 1, 1)
        pltpu.sync_copy(rows_v.at[0], acc_sh.at[idx_v.at[i0]], add=True)
        wait(i0 + 1, 1)
        start(i0 + 2, 0)
        pltpu.sync_copy(rows_v.at[1], acc_sh.at[idx_v.at[i0 + 1]], add=True)
        return ()

    lax.fori_loop(0, _NCHUNK // 2, pair, (), unroll=False)
    wait(_NCHUNK - 1, 0)
    pltpu.sync_copy(rows_v.at[0], acc_sh.at[idx_v.at[_NCHUNK - 1]], add=True)
    plsc.subcore_barrier()
    pltpu.sync_copy(acc_sh.at[pl.ds(nbase, _NPT), :],
                    out_hbm.at[c, pl.ds(nbase, _NPT), :])


@functools.lru_cache(maxsize=None)
def _sc_impls():
    mesh = plsc.VectorSubcoreMesh(core_axis_name="c", subcore_axis_name="s",
                                  num_cores=_NC, num_subcores=_NS)
    gather = pl.kernel(
        _sc_gather_body,
        out_type=jax.ShapeDtypeStruct((_E, _F), jnp.float32),
        mesh=mesh,
        scratch_types=[
            pltpu.VMEM((_NCHUNK, _CH), jnp.int32),
            pltpu.VMEM((2, _CH, _F), jnp.float32),
            pltpu.SemaphoreType.DMA,
            pltpu.SemaphoreType.DMA,
        ],
    )
    scatter = pl.kernel(
        _sc_scatter_body,
        out_type=jax.ShapeDtypeStruct((_NC, _NPAD, _F), jnp.float32),
        mesh=mesh,
        scratch_types=[
            pltpu.VMEM_SHARED((_NPAD, _F), jnp.float32),
            pltpu.VMEM((_NCHUNK, _CH), jnp.int32),
            pltpu.VMEM((2, _CH, _F), jnp.float32),
            pltpu.SemaphoreType.DMA,
            pltpu.SemaphoreType.DMA,
        ],
    )
    return gather, scatter


_CHK = 3200


def _tc_coef_body(vt_ref, radt_ref, wr1t_ref, br1t_ref, wr2t_ref, br2t_ref,
                  ct_ref):
    v = vt_ref[...]
    rsq = jnp.sum(v * v, axis=0, keepdims=True)
    inv = 1.0 / (jnp.sqrt(rsq) + 1e-8)
    x = v[0:1, :] * inv
    y = v[1:2, :] * inv
    z = v[2:3, :] * inv
    rh = jnp.dot(wr1t_ref[...], radt_ref[...],
                 preferred_element_type=jnp.float32) + br1t_ref[...]
    rh = rh * (1.0 / (1.0 + jnp.exp(-rh)))
    rw = jnp.dot(wr2t_ref[...], rh,
                 preferred_element_type=jnp.float32) + br2t_ref[...]
    yt = jnp.concatenate([jnp.ones_like(x), x, y, z,
                          x * y, y * z, 3.0 * z * z - 1.0, x * z,
                          x * x - y * y], axis=0)
    ct_ref[...] = (yt * rw).astype(jnp.bfloat16)


def _tc_coef(vt, radt, wr1t, br1t, wr2t, br2t):
    grid = (_E // _CHK,)
    return pl.pallas_call(
        _tc_coef_body,
        grid=grid,
        in_specs=[
            pl.BlockSpec((3, _CHK), lambda i: (0, i)),
            pl.BlockSpec((_NB, _CHK), lambda i: (0, i)),
            pl.BlockSpec((64, _NB), lambda i: (0, 0)),
            pl.BlockSpec((64, 1), lambda i: (0, 0)),
            pl.BlockSpec((_SHD, 64), lambda i: (0, 0)),
            pl.BlockSpec((_SHD, 1), lambda i: (0, 0)),
        ],
        out_specs=pl.BlockSpec((_SHD, _CHK), lambda i: (0, i)),
        out_shape=jax.ShapeDtypeStruct((_SHD, _E), jnp.bfloat16),
    )(vt, radt, wr1t, br1t, wr2t, br2t)


_TE = 1280


def _tc_edge_body(ct_ref, h_ref, bsel_ref, wlin_ref, m_ref):
    h_bf = h_ref[...].astype(jnp.bfloat16)
    pmat = lax.dot_general(
        ct_ref[...], bsel_ref[...],
        dimension_numbers=(((0,), (0,)), ((), ())),
        preferred_element_type=jnp.float32).astype(jnp.bfloat16)
    xedge = jnp.concatenate(
        [pmat[:, lm * _F:(lm + 1) * _F] * h_bf for lm in range(_SHD)], axis=1)
    m_ref[...] = jnp.dot(xedge, wlin_ref[...],
                         preferred_element_type=jnp.float32)


def _tc_edge(ct, h_send, bsel, wlin):
    grid = (_E // _TE,)
    return pl.pallas_call(
        _tc_edge_body,
        grid=grid,
        in_specs=[
            pl.BlockSpec((_SHD, _TE), lambda i: (0, i)),
            pl.BlockSpec((_TE, _F), lambda i: (i, 0)),
            pl.BlockSpec((_SHD, _SHD * _F), lambda i: (0, 0)),
            pl.BlockSpec((_SHD * _F, _F), lambda i: (0, 0)),
        ],
        out_specs=pl.BlockSpec((_TE, _F), lambda i: (i, 0)),
        out_shape=jax.ShapeDtypeStruct((_E, _F), jnp.float32),
    )(ct, h_send, bsel, wlin)


_TN = 1000


def _tc_node_body(p0_ref, p1_ref, nf_ref, spec_ref, wskip_ref, wprod_ref,
                  wpl_ref, wread_ref, out1_ref, feats_ref):
    f = (p0_ref[...] + p1_ref[...]) * _INV_SQRT_AVG
    spec = spec_ref[...]
    nf = nf_ref[...]
    parts = [jnp.where(spec == s, nf, 0.0) for s in range(_NSPEC)]
    xcat = jnp.concatenate(parts, axis=1)
    sc = jnp.dot(xcat, wskip_ref[...], preferred_element_type=jnp.float32)
    iota = lax.broadcasted_iota(jnp.int32, (1, _NSPEC), 1)
    onehot = (spec == iota).astype(jnp.float32)
    w = jnp.dot(onehot, wprod_ref[...], preferred_element_type=jnp.float32)
    w0, w1, w2 = w[:, :_F], w[:, _F:2 * _F], w[:, 2 * _F:3 * _F]
    pb = (w0 + w1 * f + w2 * (f * f)) * f
    feats = jnp.dot(pb, wpl_ref[...], preferred_element_type=jnp.float32) + sc
    feats_ref[...] = feats
    out1_ref[...] = jnp.dot(feats, wread_ref[...],
                            preferred_element_type=jnp.float32)


def _tc_node(p0, p1, node_feats, spec2, wskip_flat, wprod2, wpl, wread):
    grid = (_N // _TN,)
    return pl.pallas_call(
        _tc_node_body,
        grid=grid,
        in_specs=[
            pl.BlockSpec((_TN, _F), lambda i: (i, 0)),
            pl.BlockSpec((_TN, _F), lambda i: (i, 0)),
            pl.BlockSpec((_TN, _F), lambda i: (i, 0)),
            pl.BlockSpec((_TN, 1), lambda i: (i, 0)),
            pl.BlockSpec((_NSPEC * _F, _F), lambda i: (0, 0)),
            pl.BlockSpec((_NSPEC, 3 * _F), lambda i: (0, 0)),
            pl.BlockSpec((_F, _F), lambda i: (0, 0)),
            pl.BlockSpec((_F, 1), lambda i: (0, 0)),
        ],
        out_specs=[
            pl.BlockSpec((_TN, 1), lambda i: (i, 0)),
            pl.BlockSpec((_TN, _F), lambda i: (i, 0)),
        ],
        out_shape=[
            jax.ShapeDtypeStruct((_N, 1), jnp.float32),
            jax.ShapeDtypeStruct((_N, _F), jnp.float32),
        ],
    )(p0, p1, node_feats, spec2, wskip_flat, wprod2, wpl, wread)


def kernel(vectors, node_feats, node_specie, radial_embedding, senders,
           receivers, W_skip, Wr1, br1, Wr2, br2, W_lin, w_prod, W_prodlin,
           W_read):
    snd3 = senders.astype(jnp.int32).reshape(_NW, _NCHUNK, _CH)
    rcv3 = receivers.astype(jnp.int32).reshape(_NW, _NCHUNK, _CH)
    wlin_bf = W_lin.astype(jnp.bfloat16)
    bsel = jnp.repeat(jnp.eye(_SHD, dtype=jnp.bfloat16), _F, axis=1)
    zeros_tile = jnp.zeros((_NPT, _F), jnp.float32)

    sc_gather, sc_scatter = _sc_impls()
    h_send = sc_gather(node_feats, snd3)
    ct = _tc_coef(vectors.T, radial_embedding.T, Wr1.T,
                  br1.reshape(64, 1), Wr2.T, br2.reshape(_SHD, 1))
    m = _tc_edge(ct, h_send, bsel, wlin_bf)
    partials = sc_scatter(m, rcv3, zeros_tile)
    p0 = partials[0, :_N]
    p1 = partials[1, :_N]

    spec2 = node_specie.astype(jnp.int32).reshape(_N, 1)
    wskip_flat = W_skip.reshape(_NSPEC * _F, _F)
    wprod2 = w_prod.reshape(_NSPEC, 3 * _F)
    node_outputs, feats = _tc_node(p0, p1, node_feats, spec2,
                                   wskip_flat, wprod2, W_prodlin, W_read)
    return node_outputs, feats


# speedup vs baseline: 1.6394x; 1.0355x over previous
import functools

import jax
import jax.numpy as jnp
from jax import lax
from jax.experimental import pallas as pl
from jax.experimental.pallas import tpu as pltpu
from jax.experimental.pallas import tpu_sc as plsc

_N = 10000
_E = 160000
_F = 128
_NB = 8
_SHD = 9
_NSPEC = 10
_INV_SQRT_AVG = 0.25

_NC = 2
_NS = 16
_NW = _NC * _NS
_EPW = _E // _NW
_CH = 40
_NCHUNK = _EPW // _CH
_NPAD = 10240
_NPT = _NPAD // _NS

def _sc_gather_body(nf_hbm, snd3_hbm, out_hbm, idx_v, rows_v, sem0, sem1):
    c = lax.axis_index("c")
    s = lax.axis_index("s")
    wid = c * _NS + s
    base0 = wid * _EPW
    pltpu.sync_copy(snd3_hbm.at[wid], idx_v)
    sems = (sem0, sem1)

    def start(i, b):
        pltpu.async_copy(nf_hbm.at[idx_v.at[i]], rows_v.at[b], sems[b])

    def finish(i, b):
        pltpu.make_async_copy(nf_hbm.at[idx_v.at[i]], rows_v.at[b],
                              sems[b]).wait()
        pltpu.sync_copy(rows_v.at[b],
                        out_hbm.at[pl.ds(base0 + i * _CH, _CH), :])

    start(0, 0)

    def pair(j, _):
        i0 = j * 2
        pltpu.make_async_copy(nf_hbm.at[idx_v.at[i0]], rows_v.at[0],
                              sem0).wait()
        start(i0 + 1, 1)
        pltpu.sync_copy(rows_v.at[0],
                        out_hbm.at[pl.ds(base0 + i0 * _CH, _CH), :])
        pltpu.make_async_copy(nf_hbm.at[idx_v.at[i0 + 1]], rows_v.at[1],
                              sem1).wait()
        start(i0 + 2, 0)
        pltpu.sync_copy(rows_v.at[1],
                        out_hbm.at[pl.ds(base0 + (i0 + 1) * _CH, _CH), :])
        return ()

    lax.fori_loop(0, _NCHUNK // 2, pair, (), unroll=False)
    finish(_NCHUNK - 1, 0)


def _sc_scatter_body(m_hbm, rcv3_hbm, zeros_hbm, out_hbm, acc_sh, idx_v,
                     rows_v, sem0, sem1):
    c = lax.axis_index("c")
    s = lax.axis_index("s")
    wid = c * _NS + s
    base0 = wid * _EPW
    nbase = s * _NPT
    pltpu.sync_copy(zeros_hbm, acc_sh.at[pl.ds(nbase, _NPT), :])
    pltpu.sync_copy(rcv3_hbm.at[wid], idx_v)
    plsc.subcore_barrier()

    sems = (sem0, sem1)

    def start(i, b):
        pltpu.async_copy(m_hbm.at[pl.ds(base0 + i * _CH, _CH), :],
                         rows_v.at[b], sems[b])

    def wait(i, b):
        pltpu.make_async_copy(m_hbm.at[pl.ds(base0 + i * _CH, _CH), :],
                              rows_v.at[b], sems[b]).wait()

    start(0, 0)

    def pair(j, _):
        i0 = j * 2
        wait(i0, 0)
        start(i0 + 1, 1)
        pltpu.sync_copy(rows_v.at[0], acc_sh.at[idx_v.at[i0]], add=True)
        wait(i0 + 1, 1)
        start(i0 + 2, 0)
        pltpu.sync_copy(rows_v.at[1], acc_sh.at[idx_v.at[i0 + 1]], add=True)
        return ()

    lax.fori_loop(0, _NCHUNK // 2, pair, (), unroll=False)
    wait(_NCHUNK - 1, 0)
    pltpu.sync_copy(rows_v.at[0], acc_sh.at[idx_v.at[_NCHUNK - 1]], add=True)
    plsc.subcore_barrier()
    pltpu.sync_copy(acc_sh.at[pl.ds(nbase, _NPT), :],
                    out_hbm.at[c, pl.ds(nbase, _NPT), :])


@functools.lru_cache(maxsize=None)
def _sc_impls():
    mesh = plsc.VectorSubcoreMesh(core_axis_name="c", subcore_axis_name="s",
                                  num_cores=_NC, num_subcores=_NS)
    gather = pl.kernel(
        _sc_gather_body,
        out_type=jax.ShapeDtypeStruct((_E, _F), jnp.float32),
        mesh=mesh,
        scratch_types=[
            pltpu.VMEM((_NCHUNK, _CH), jnp.int32),
            pltpu.VMEM((2, _CH, _F), jnp.float32),
            pltpu.SemaphoreType.DMA,
            pltpu.SemaphoreType.DMA,
        ],
    )
    scatter = pl.kernel(
        _sc_scatter_body,
        out_type=jax.ShapeDtypeStruct((_NC, _NPAD, _F), jnp.float32),
        mesh=mesh,
        scratch_types=[
            pltpu.VMEM_SHARED((_NPAD, _F), jnp.float32),
            pltpu.VMEM((_NCHUNK, _CH), jnp.int32),
            pltpu.VMEM((2, _CH, _F), jnp.float32),
            pltpu.SemaphoreType.DMA,
            pltpu.SemaphoreType.DMA,
        ],
    )
    return gather, scatter


_CHK = 3200


def _tc_coef_body(vt_ref, radt_ref, wr1t_ref, br1t_ref, wr2t_ref, br2t_ref,
                  ct_ref):
    v = vt_ref[...]
    rsq = jnp.sum(v * v, axis=0, keepdims=True)
    inv = 1.0 / (jnp.sqrt(rsq) + 1e-8)
    x = v[0:1, :] * inv
    y = v[1:2, :] * inv
    z = v[2:3, :] * inv
    rh = jnp.dot(wr1t_ref[...], radt_ref[...],
                 preferred_element_type=jnp.float32) + br1t_ref[...]
    rh = rh * (1.0 / (1.0 + jnp.exp(-rh)))
    rw = jnp.dot(wr2t_ref[...], rh,
                 preferred_element_type=jnp.float32) + br2t_ref[...]
    yt = jnp.concatenate([jnp.ones_like(x), x, y, z,
                          x * y, y * z, 3.0 * z * z - 1.0, x * z,
                          x * x - y * y], axis=0)
    ct_ref[...] = (yt * rw).astype(jnp.bfloat16)


def _tc_coef(vt, radt, wr1t, br1t, wr2t, br2t):
    grid = (_E // _CHK,)
    return pl.pallas_call(
        _tc_coef_body,
        grid=grid,
        in_specs=[
            pl.BlockSpec((3, _CHK), lambda i: (0, i)),
            pl.BlockSpec((_NB, _CHK), lambda i: (0, i)),
            pl.BlockSpec((64, _NB), lambda i: (0, 0)),
            pl.BlockSpec((64, 1), lambda i: (0, 0)),
            pl.BlockSpec((_SHD, 64), lambda i: (0, 0)),
            pl.BlockSpec((_SHD, 1), lambda i: (0, 0)),
        ],
        out_specs=pl.BlockSpec((_SHD, _CHK), lambda i: (0, i)),
        out_shape=jax.ShapeDtypeStruct((_SHD, _E), jnp.bfloat16),
    )(vt, radt, wr1t, br1t, wr2t, br2t)


_TE = 3200


def _tc_edge_body(ct_ref, h_ref, bsel_ref, wlin_ref, m_ref):
    h_bf = h_ref[...].astype(jnp.bfloat16)
    pmat = lax.dot_general(
        ct_ref[...], bsel_ref[...],
        dimension_numbers=(((0,), (0,)), ((), ())),
        preferred_element_type=jnp.float32).astype(jnp.bfloat16)
    xedge = jnp.concatenate(
        [pmat[:, lm * _F:(lm + 1) * _F] * h_bf for lm in range(_SHD)], axis=1)
    m_ref[...] = jnp.dot(xedge, wlin_ref[...],
                         preferred_element_type=jnp.float32)


def _tc_edge(ct, h_send, bsel, wlin):
    grid = (_E // _TE,)
    return pl.pallas_call(
        _tc_edge_body,
        grid=grid,
        in_specs=[
            pl.BlockSpec((_SHD, _TE), lambda i: (0, i)),
            pl.BlockSpec((_TE, _F), lambda i: (i, 0)),
            pl.BlockSpec((_SHD, _SHD * _F), lambda i: (0, 0)),
            pl.BlockSpec((_SHD * _F, _F), lambda i: (0, 0)),
        ],
        out_specs=pl.BlockSpec((_TE, _F), lambda i: (i, 0)),
        out_shape=jax.ShapeDtypeStruct((_E, _F), jnp.float32),
    )(ct, h_send, bsel, wlin)


_TN = 1000


def _tc_node_body(p0_ref, p1_ref, nf_ref, spec_ref, wskip_ref, wprod_ref,
                  wpl_ref, wread_ref, out1_ref, feats_ref):
    f = (p0_ref[...] + p1_ref[...]) * _INV_SQRT_AVG
    spec = spec_ref[...]
    nf = nf_ref[...]
    parts = [jnp.where(spec == s, nf, 0.0) for s in range(_NSPEC)]
    xcat = jnp.concatenate(parts, axis=1)
    sc = jnp.dot(xcat, wskip_ref[...], preferred_element_type=jnp.float32)
    iota = lax.broadcasted_iota(jnp.int32, (1, _NSPEC), 1)
    onehot = (spec == iota).astype(jnp.float32)
    w = jnp.dot(onehot, wprod_ref[...], preferred_element_type=jnp.float32)
    w0, w1, w2 = w[:, :_F], w[:, _F:2 * _F], w[:, 2 * _F:3 * _F]
    pb = (w0 + w1 * f + w2 * (f * f)) * f
    feats = jnp.dot(pb, wpl_ref[...], preferred_element_type=jnp.float32) + sc
    feats_ref[...] = feats
    out1_ref[...] = jnp.dot(feats, wread_ref[...],
                            preferred_element_type=jnp.float32)


def _tc_node(p0, p1, node_feats, spec2, wskip_flat, wprod2, wpl, wread):
    grid = (_N // _TN,)
    return pl.pallas_call(
        _tc_node_body,
        grid=grid,
        in_specs=[
            pl.BlockSpec((_TN, _F), lambda i: (i, 0)),
            pl.BlockSpec((_TN, _F), lambda i: (i, 0)),
            pl.BlockSpec((_TN, _F), lambda i: (i, 0)),
            pl.BlockSpec((_TN, 1), lambda i: (i, 0)),
            pl.BlockSpec((_NSPEC * _F, _F), lambda i: (0, 0)),
            pl.BlockSpec((_NSPEC, 3 * _F), lambda i: (0, 0)),
            pl.BlockSpec((_F, _F), lambda i: (0, 0)),
            pl.BlockSpec((_F, 1), lambda i: (0, 0)),
        ],
        out_specs=[
            pl.BlockSpec((_TN, 1), lambda i: (i, 0)),
            pl.BlockSpec((_TN, _F), lambda i: (i, 0)),
        ],
        out_shape=[
            jax.ShapeDtypeStruct((_N, 1), jnp.float32),
            jax.ShapeDtypeStruct((_N, _F), jnp.float32),
        ],
    )(p0, p1, node_feats, spec2, wskip_flat, wprod2, wpl, wread)


def kernel(vectors, node_feats, node_specie, radial_embedding, senders,
           receivers, W_skip, Wr1, br1, Wr2, br2, W_lin, w_prod, W_prodlin,
           W_read):
    snd3 = senders.astype(jnp.int32).reshape(_NW, _NCHUNK, _CH)
    rcv3 = receivers.astype(jnp.int32).reshape(_NW, _NCHUNK, _CH)
    wlin_bf = W_lin.astype(jnp.bfloat16)
    bsel = jnp.repeat(jnp.eye(_SHD, dtype=jnp.bfloat16), _F, axis=1)
    zeros_tile = jnp.zeros((_NPT, _F), jnp.float32)

    sc_gather, sc_scatter = _sc_impls()
    h_send = sc_gather(node_feats, snd3)
    ct = _tc_coef(vectors.T, radial_embedding.T, Wr1.T,
                  br1.reshape(64, 1), Wr2.T, br2.reshape(_SHD, 1))
    m = _tc_edge(ct, h_send, bsel, wlin_bf)
    partials = sc_scatter(m, rcv3, zeros_tile)
    p0 = partials[0, :_N]
    p1 = partials[1, :_N]

    spec2 = node_specie.astype(jnp.int32).reshape(_N, 1)
    wskip_flat = W_skip.reshape(_NSPEC * _F, _F)
    wprod2 = w_prod.reshape(_NSPEC, 3 * _F)
    node_outputs, feats = _tc_node(p0, p1, node_feats, spec2,
                                   wskip_flat, wprod2, W_prodlin, W_read)
    return node_outputs, feats


# speedup vs baseline: 2.1567x; 1.3155x over previous
import functools

import jax
import jax.numpy as jnp
from jax import lax
from jax.experimental import pallas as pl
from jax.experimental.pallas import tpu as pltpu
from jax.experimental.pallas import tpu_sc as plsc

_N = 10000
_E = 160000
_F = 128
_NB = 8
_SHD = 9
_NSPEC = 10
_INV_SQRT_AVG = 0.25

_NC = 2
_NS = 16
_NW = _NC * _NS
_EPW = _E // _NW
_CH = 40
_NCHUNK = _EPW // _CH
_NPAD = 10240
_NPT = _NPAD // _NS

def _sc_gather_body(nf_hbm, snd3_hbm, out_hbm, idx_v, rows_v, sem0, sem1,
                    sem2, sem3):
    c = lax.axis_index("c")
    s = lax.axis_index("s")
    wid = c * _NS + s
    base0 = wid * _EPW
    pltpu.sync_copy(snd3_hbm.at[wid], idx_v)
    sems = (sem0, sem1, sem2, sem3)

    def start(i, b):
        pltpu.async_copy(nf_hbm.at[idx_v.at[i]], rows_v.at[b], sems[b])

    def wait(i, b):
        pltpu.make_async_copy(nf_hbm.at[idx_v.at[i]], rows_v.at[b],
                              sems[b]).wait()

    def out(i, b):
        pltpu.sync_copy(rows_v.at[b],
                        out_hbm.at[pl.ds(base0 + i * _CH, _CH), :])

    for b in range(3):
        start(b, b)

    def quad(j, _):
        for b in range(4):
            i = j * 4 + b
            wait(i, b)

            @pl.when(i + 3 < _NCHUNK)
            def _():
                start(i + 3, (b + 3) % 4)

            out(i, b)
        return ()

    lax.fori_loop(0, _NCHUNK // 4, quad, (), unroll=False)
    wait(_NCHUNK - 1, (_NCHUNK - 1) % 4)
    out(_NCHUNK - 1, (_NCHUNK - 1) % 4)


def _sc_scatter_body(m_hbm, rcv3_hbm, zeros_hbm, out_hbm, acc_sh, idx_v,
                     rows_v, sem0, sem1, sem2, sem3):
    c = lax.axis_index("c")
    s = lax.axis_index("s")
    wid = c * _NS + s
    base0 = wid * _EPW
    nbase = s * _NPT
    pltpu.sync_copy(zeros_hbm, acc_sh.at[pl.ds(nbase, _NPT), :])
    pltpu.sync_copy(rcv3_hbm.at[wid], idx_v)
    plsc.subcore_barrier()

    sems = (sem0, sem1, sem2, sem3)

    def start(i, b):
        pltpu.async_copy(m_hbm.at[pl.ds(base0 + i * _CH, _CH), :],
                         rows_v.at[b], sems[b])

    def wait(i, b):
        pltpu.make_async_copy(m_hbm.at[pl.ds(base0 + i * _CH, _CH), :],
                              rows_v.at[b], sems[b]).wait()

    for b in range(3):
        start(b, b)

    def quad(j, _):
        for b in range(4):
            i = j * 4 + b
            wait(i, b)

            @pl.when(i + 3 < _NCHUNK)
            def _():
                start(i + 3, (b + 3) % 4)

            pltpu.sync_copy(rows_v.at[b], acc_sh.at[idx_v.at[i]], add=True)
        return ()

    lax.fori_loop(0, _NCHUNK // 4, quad, (), unroll=False)
    wait(_NCHUNK - 1, (_NCHUNK - 1) % 4)
    pltpu.sync_copy(rows_v.at[(_NCHUNK - 1) % 4],
                    acc_sh.at[idx_v.at[_NCHUNK - 1]], add=True)
    plsc.subcore_barrier()
    pltpu.sync_copy(acc_sh.at[pl.ds(nbase, _NPT), :],
                    out_hbm.at[c, pl.ds(nbase, _NPT), :])


@functools.lru_cache(maxsize=None)
def _sc_impls():
    mesh = plsc.VectorSubcoreMesh(core_axis_name="c", subcore_axis_name="s",
                                  num_cores=_NC, num_subcores=_NS)
    gather = pl.kernel(
        _sc_gather_body,
        out_type=jax.ShapeDtypeStruct((_E, _F), jnp.float32),
        mesh=mesh,
        scratch_types=[
            pltpu.VMEM((_NCHUNK, _CH), jnp.int32),
            pltpu.VMEM((4, _CH, _F), jnp.float32),
            pltpu.SemaphoreType.DMA,
            pltpu.SemaphoreType.DMA,
            pltpu.SemaphoreType.DMA,
            pltpu.SemaphoreType.DMA,
        ],
    )
    scatter = pl.kernel(
        _sc_scatter_body,
        out_type=jax.ShapeDtypeStruct((_NC, _NPAD, _F), jnp.float32),
        mesh=mesh,
        scratch_types=[
            pltpu.VMEM_SHARED((_NPAD, _F), jnp.float32),
            pltpu.VMEM((_NCHUNK, _CH), jnp.int32),
            pltpu.VMEM((4, _CH, _F), jnp.float32),
            pltpu.SemaphoreType.DMA,
            pltpu.SemaphoreType.DMA,
            pltpu.SemaphoreType.DMA,
            pltpu.SemaphoreType.DMA,
        ],
    )
    return gather, scatter


_CHK = 3200


def _tc_coef_body(vt_ref, radt_ref, wr1t_ref, br1t_ref, wr2t_ref, br2t_ref,
                  ct_ref):
    v = vt_ref[...]
    rsq = jnp.sum(v * v, axis=0, keepdims=True)
    inv = 1.0 / (jnp.sqrt(rsq) + 1e-8)
    x = v[0:1, :] * inv
    y = v[1:2, :] * inv
    z = v[2:3, :] * inv
    rh = jnp.dot(wr1t_ref[...], radt_ref[...],
                 preferred_element_type=jnp.float32) + br1t_ref[...]
    rh = rh * (1.0 / (1.0 + jnp.exp(-rh)))
    rw = jnp.dot(wr2t_ref[...], rh,
                 preferred_element_type=jnp.float32) + br2t_ref[...]
    yt = jnp.concatenate([jnp.ones_like(x), x, y, z,
                          x * y, y * z, 3.0 * z * z - 1.0, x * z,
                          x * x - y * y], axis=0)
    ct_ref[...] = (yt * rw).astype(jnp.bfloat16)


def _tc_coef(vt, radt, wr1t, br1t, wr2t, br2t):
    grid = (_E // _CHK,)
    return pl.pallas_call(
        _tc_coef_body,
        grid=grid,
        in_specs=[
            pl.BlockSpec((3, _CHK), lambda i: (0, i)),
            pl.BlockSpec((_NB, _CHK), lambda i: (0, i)),
            pl.BlockSpec((64, _NB), lambda i: (0, 0)),
            pl.BlockSpec((64, 1), lambda i: (0, 0)),
            pl.BlockSpec((_SHD, 64), lambda i: (0, 0)),
            pl.BlockSpec((_SHD, 1), lambda i: (0, 0)),
        ],
        out_specs=pl.BlockSpec((_SHD, _CHK), lambda i: (0, i)),
        out_shape=jax.ShapeDtypeStruct((_SHD, _E), jnp.bfloat16),
    )(vt, radt, wr1t, br1t, wr2t, br2t)


_TE = 3200


def _tc_edge_body(ct_ref, h_ref, bsel_ref, wlin_ref, m_ref):
    h_bf = h_ref[...].astype(jnp.bfloat16)
    pmat = lax.dot_general(
        ct_ref[...], bsel_ref[...],
        dimension_numbers=(((0,), (0,)), ((), ())),
        preferred_element_type=jnp.float32).astype(jnp.bfloat16)
    xedge = jnp.concatenate(
        [pmat[:, lm * _F:(lm + 1) * _F] * h_bf for lm in range(_SHD)], axis=1)
    m_ref[...] = jnp.dot(xedge, wlin_ref[...],
                         preferred_element_type=jnp.float32)


def _tc_edge(ct, h_send, bsel, wlin):
    grid = (_E // _TE,)
    return pl.pallas_call(
        _tc_edge_body,
        grid=grid,
        in_specs=[
            pl.BlockSpec((_SHD, _TE), lambda i: (0, i)),
            pl.BlockSpec((_TE, _F), lambda i: (i, 0)),
            pl.BlockSpec((_SHD, _SHD * _F), lambda i: (0, 0)),
            pl.BlockSpec((_SHD * _F, _F), lambda i: (0, 0)),
        ],
        out_specs=pl.BlockSpec((_TE, _F), lambda i: (i, 0)),
        out_shape=jax.ShapeDtypeStruct((_E, _F), jnp.float32),
    )(ct, h_send, bsel, wlin)


_TN = 1000


def _tc_node_body(p0_ref, p1_ref, nf_ref, spec_ref, wskip_ref, wprod_ref,
                  wpl_ref, wread_ref, out1_ref, feats_ref):
    f = (p0_ref[...] + p1_ref[...]) * _INV_SQRT_AVG
    spec = spec_ref[...]
    nf = nf_ref[...]
    parts = [jnp.where(spec == s, nf, 0.0) for s in range(_NSPEC)]
    xcat = jnp.concatenate(parts, axis=1)
    sc = jnp.dot(xcat, wskip_ref[...], preferred_element_type=jnp.float32)
    iota = lax.broadcasted_iota(jnp.int32, (1, _NSPEC), 1)
    onehot = (spec == iota).astype(jnp.float32)
    w = jnp.dot(onehot, wprod_ref[...], preferred_element_type=jnp.float32)
    w0, w1, w2 = w[:, :_F], w[:, _F:2 * _F], w[:, 2 * _F:3 * _F]
    pb = (w0 + w1 * f + w2 * (f * f)) * f
    feats = jnp.dot(pb, wpl_ref[...], preferred_element_type=jnp.float32) + sc
    feats_ref[...] = feats
    out1_ref[...] = jnp.dot(feats, wread_ref[...],
                            preferred_element_type=jnp.float32)


def _tc_node(p0, p1, node_feats, spec2, wskip_flat, wprod2, wpl, wread):
    grid = (_N // _TN,)
    return pl.pallas_call(
        _tc_node_body,
        grid=grid,
        in_specs=[
            pl.BlockSpec((_TN, _F), lambda i: (i, 0)),
            pl.BlockSpec((_TN, _F), lambda i: (i, 0)),
            pl.BlockSpec((_TN, _F), lambda i: (i, 0)),
            pl.BlockSpec((_TN, 1), lambda i: (i, 0)),
            pl.BlockSpec((_NSPEC * _F, _F), lambda i: (0, 0)),
            pl.BlockSpec((_NSPEC, 3 * _F), lambda i: (0, 0)),
            pl.BlockSpec((_F, _F), lambda i: (0, 0)),
            pl.BlockSpec((_F, 1), lambda i: (0, 0)),
        ],
        out_specs=[
            pl.BlockSpec((_TN, 1), lambda i: (i, 0)),
            pl.BlockSpec((_TN, _F), lambda i: (i, 0)),
        ],
        out_shape=[
            jax.ShapeDtypeStruct((_N, 1), jnp.float32),
            jax.ShapeDtypeStruct((_N, _F), jnp.float32),
        ],
    )(p0, p1, node_feats, spec2, wskip_flat, wprod2, wpl, wread)


def kernel(vectors, node_feats, node_specie, radial_embedding, senders,
           receivers, W_skip, Wr1, br1, Wr2, br2, W_lin, w_prod, W_prodlin,
           W_read):
    snd3 = senders.astype(jnp.int32).reshape(_NW, _NCHUNK, _CH)
    rcv3 = receivers.astype(jnp.int32).reshape(_NW, _NCHUNK, _CH)
    wlin_bf = W_lin.astype(jnp.bfloat16)
    bsel = jnp.repeat(jnp.eye(_SHD, dtype=jnp.bfloat16), _F, axis=1)
    zeros_tile = jnp.zeros((_NPT, _F), jnp.float32)

    sc_gather, sc_scatter = _sc_impls()
    h_send = sc_gather(node_feats, snd3)
    ct = _tc_coef(vectors.T, radial_embedding.T, Wr1.T,
                  br1.reshape(64, 1), Wr2.T, br2.reshape(_SHD, 1))
    m = _tc_edge(ct, h_send, bsel, wlin_bf)
    partials = sc_scatter(m, rcv3, zeros_tile)
    p0 = partials[0, :_N]
    p1 = partials[1, :_N]

    spec2 = node_specie.astype(jnp.int32).reshape(_N, 1)
    wskip_flat = W_skip.reshape(_NSPEC * _F, _F)
    wprod2 = w_prod.reshape(_NSPEC, 3 * _F)
    node_outputs, feats = _tc_node(p0, p1, node_feats, spec2,
                                   wskip_flat, wprod2, W_prodlin, W_read)
    return node_outputs, feats


# speedup vs baseline: 2.2376x; 1.0375x over previous
import functools

import jax
import jax.numpy as jnp
from jax import lax
from jax.experimental import pallas as pl
from jax.experimental.pallas import tpu as pltpu
from jax.experimental.pallas import tpu_sc as plsc

_N = 10000
_E = 160000
_F = 128
_NB = 8
_SHD = 9
_NSPEC = 10
_INV_SQRT_AVG = 0.25

_NC = 2
_NS = 16
_NW = _NC * _NS
_EPW = _E // _NW
_CH = 40
_NCHUNK = _EPW // _CH
_EA = 102400
_NPAD = 10240
_NPT = _NPAD // _NS

def _make_gather_body(epw, nchunk):
    def body_fn(nf_hbm, snd3_hbm, out_hbm, idx_v, rows_v, sem0, sem1, sem2,
                sem3):
        c = lax.axis_index("c")
        s = lax.axis_index("s")
        wid = c * _NS + s
        base0 = wid * epw
        pltpu.sync_copy(snd3_hbm.at[wid], idx_v)
        sems = (sem0, sem1, sem2, sem3)

        def start(i, b):
            pltpu.async_copy(nf_hbm.at[idx_v.at[i]], rows_v.at[b], sems[b])

        def wait(i, b):
            pltpu.make_async_copy(nf_hbm.at[idx_v.at[i]], rows_v.at[b],
                                  sems[b]).wait()

        def out(i, b):
            pltpu.sync_copy(rows_v.at[b],
                            out_hbm.at[pl.ds(base0 + i * _CH, _CH), :])

        for b in range(3):
            start(b, b)

        def quad(j, _):
            for b in range(4):
                i = j * 4 + b
                wait(i, b)

                @pl.when(i + 3 < nchunk)
                def _():
                    start(i + 3, (b + 3) % 4)

                out(i, b)
            return ()

        lax.fori_loop(0, nchunk // 4, quad, (), unroll=False)
        for i in range((nchunk // 4) * 4, nchunk):
            wait(i, i % 4)
            out(i, i % 4)

    return body_fn


def _make_scatter_body(epw, nchunk):
    def body_fn(m_hbm, rcv3_hbm, zeros_hbm, out_hbm, acc_sh, idx_v, rows_v,
                sem0, sem1, sem2, sem3):
        c = lax.axis_index("c")
        s = lax.axis_index("s")
        wid = c * _NS + s
        base0 = wid * epw
        nbase = s * _NPT
        pltpu.sync_copy(zeros_hbm, acc_sh.at[pl.ds(nbase, _NPT), :])
        pltpu.sync_copy(rcv3_hbm.at[wid], idx_v)
        plsc.subcore_barrier()
        sems = (sem0, sem1, sem2, sem3)

        def start(i, b):
            pltpu.async_copy(m_hbm.at[pl.ds(base0 + i * _CH, _CH), :],
                             rows_v.at[b], sems[b])

        def wait(i, b):
            pltpu.make_async_copy(m_hbm.at[pl.ds(base0 + i * _CH, _CH), :],
                                  rows_v.at[b], sems[b]).wait()

        for b in range(3):
            start(b, b)

        def quad(j, _):
            for b in range(4):
                i = j * 4 + b
                wait(i, b)

                @pl.when(i + 3 < nchunk)
                def _():
                    start(i + 3, (b + 3) % 4)

                pltpu.sync_copy(rows_v.at[b], acc_sh.at[idx_v.at[i]],
                                add=True)
            return ()

        lax.fori_loop(0, nchunk // 4, quad, (), unroll=False)
        for i in range((nchunk // 4) * 4, nchunk):
            wait(i, i % 4)
            pltpu.sync_copy(rows_v.at[i % 4], acc_sh.at[idx_v.at[i]],
                            add=True)
        plsc.subcore_barrier()
        pltpu.sync_copy(acc_sh.at[pl.ds(nbase, _NPT), :],
                        out_hbm.at[c, pl.ds(nbase, _NPT), :])

    return body_fn


@functools.lru_cache(maxsize=None)
def _sc_impls(esize):
    epw = esize // _NW
    nchunk = epw // _CH
    mesh = plsc.VectorSubcoreMesh(core_axis_name="c", subcore_axis_name="s",
                                  num_cores=_NC, num_subcores=_NS)
    gather = pl.kernel(
        _make_gather_body(epw, nchunk),
        out_type=jax.ShapeDtypeStruct((esize, _F), jnp.float32),
        mesh=mesh,
        scratch_types=[
            pltpu.VMEM((nchunk, _CH), jnp.int32),
            pltpu.VMEM((4, _CH, _F), jnp.float32),
            pltpu.SemaphoreType.DMA,
            pltpu.SemaphoreType.DMA,
            pltpu.SemaphoreType.DMA,
            pltpu.SemaphoreType.DMA,
        ],
    )
    scatter = pl.kernel(
        _make_scatter_body(epw, nchunk),
        out_type=jax.ShapeDtypeStruct((_NC, _NPAD, _F), jnp.float32),
        mesh=mesh,
        scratch_types=[
            pltpu.VMEM_SHARED((_NPAD, _F), jnp.float32),
            pltpu.VMEM((nchunk, _CH), jnp.int32),
            pltpu.VMEM((4, _CH, _F), jnp.float32),
            pltpu.SemaphoreType.DMA,
            pltpu.SemaphoreType.DMA,
            pltpu.SemaphoreType.DMA,
            pltpu.SemaphoreType.DMA,
        ],
    )
    return gather, scatter


_CHK = 3200


def _tc_coef_body(vt_ref, radt_ref, wr1t_ref, br1t_ref, wr2t_ref, br2t_ref,
                  ct_ref):
    v = vt_ref[...]
    rsq = jnp.sum(v * v, axis=0, keepdims=True)
    inv = 1.0 / (jnp.sqrt(rsq) + 1e-8)
    x = v[0:1, :] * inv
    y = v[1:2, :] * inv
    z = v[2:3, :] * inv
    rh = jnp.dot(wr1t_ref[...], radt_ref[...],
                 preferred_element_type=jnp.float32) + br1t_ref[...]
    rh = rh * (1.0 / (1.0 + jnp.exp(-rh)))
    rw = jnp.dot(wr2t_ref[...], rh,
                 preferred_element_type=jnp.float32) + br2t_ref[...]
    yt = jnp.concatenate([jnp.ones_like(x), x, y, z,
                          x * y, y * z, 3.0 * z * z - 1.0, x * z,
                          x * x - y * y], axis=0)
    ct_ref[...] = (yt * rw).astype(jnp.bfloat16)


def _tc_coef(vt, radt, wr1t, br1t, wr2t, br2t):
    grid = (_E // _CHK,)
    return pl.pallas_call(
        _tc_coef_body,
        grid=grid,
        in_specs=[
            pl.BlockSpec((3, _CHK), lambda i: (0, i)),
            pl.BlockSpec((_NB, _CHK), lambda i: (0, i)),
            pl.BlockSpec((64, _NB), lambda i: (0, 0)),
            pl.BlockSpec((64, 1), lambda i: (0, 0)),
            pl.BlockSpec((_SHD, 64), lambda i: (0, 0)),
            pl.BlockSpec((_SHD, 1), lambda i: (0, 0)),
        ],
        out_specs=pl.BlockSpec((_SHD, _CHK), lambda i: (0, i)),
        out_shape=jax.ShapeDtypeStruct((_SHD, _E), jnp.bfloat16),
    )(vt, radt, wr1t, br1t, wr2t, br2t)


_TE = 3200


def _tc_edge_body(ct_ref, h_ref, bsel_ref, wlin_ref, m_ref):
    h_bf = h_ref[...].astype(jnp.bfloat16)
    pmat = lax.dot_general(
        ct_ref[...], bsel_ref[...],
        dimension_numbers=(((0,), (0,)), ((), ())),
        preferred_element_type=jnp.float32).astype(jnp.bfloat16)
    xedge = jnp.concatenate(
        [pmat[:, lm * _F:(lm + 1) * _F] * h_bf for lm in range(_SHD)], axis=1)
    m_ref[...] = jnp.dot(xedge, wlin_ref[...],
                         preferred_element_type=jnp.float32)


def _tc_edge(ct, h_send, bsel, wlin, esize, blk0):
    grid = (esize // _TE,)
    return pl.pallas_call(
        _tc_edge_body,
        grid=grid,
        in_specs=[
            pl.BlockSpec((_SHD, _TE), lambda i: (0, i + blk0)),
            pl.BlockSpec((_TE, _F), lambda i: (i, 0)),
            pl.BlockSpec((_SHD, _SHD * _F), lambda i: (0, 0)),
            pl.BlockSpec((_SHD * _F, _F), lambda i: (0, 0)),
        ],
        out_specs=pl.BlockSpec((_TE, _F), lambda i: (i, 0)),
        out_shape=jax.ShapeDtypeStruct((esize, _F), jnp.float32),
    )(ct, h_send, bsel, wlin)


_TN = 1000


def _tc_node_body(p0_ref, p1_ref, p2_ref, p3_ref, nf_ref, spec_ref,
                  wskip_ref, wprod_ref, wpl_ref, wread_ref, out1_ref,
                  feats_ref):
    f = ((p0_ref[...] + p1_ref[...]) + (p2_ref[...] + p3_ref[...])) \
        * _INV_SQRT_AVG
    spec = spec_ref[...]
    nf = nf_ref[...]
    parts = [jnp.where(spec == s, nf, 0.0) for s in range(_NSPEC)]
    xcat = jnp.concatenate(parts, axis=1)
    sc = jnp.dot(xcat, wskip_ref[...], preferred_element_type=jnp.float32)
    iota = lax.broadcasted_iota(jnp.int32, (1, _NSPEC), 1)
    onehot = (spec == iota).astype(jnp.float32)
    w = jnp.dot(onehot, wprod_ref[...], preferred_element_type=jnp.float32)
    w0, w1, w2 = w[:, :_F], w[:, _F:2 * _F], w[:, 2 * _F:3 * _F]
    pb = (w0 + w1 * f + w2 * (f * f)) * f
    feats = jnp.dot(pb, wpl_ref[...], preferred_element_type=jnp.float32) + sc
    feats_ref[...] = feats
    out1_ref[...] = jnp.dot(feats, wread_ref[...],
                            preferred_element_type=jnp.float32)


def _tc_node(p0, p1, p2, p3, node_feats, spec2, wskip_flat, wprod2, wpl,
             wread):
    grid = (_N // _TN,)
    return pl.pallas_call(
        _tc_node_body,
        grid=grid,
        in_specs=[
            pl.BlockSpec((_TN, _F), lambda i: (i, 0)),
            pl.BlockSpec((_TN, _F), lambda i: (i, 0)),
            pl.BlockSpec((_TN, _F), lambda i: (i, 0)),
            pl.BlockSpec((_TN, _F), lambda i: (i, 0)),
            pl.BlockSpec((_TN, _F), lambda i: (i, 0)),
            pl.BlockSpec((_TN, 1), lambda i: (i, 0)),
            pl.BlockSpec((_NSPEC * _F, _F), lambda i: (0, 0)),
            pl.BlockSpec((_NSPEC, 3 * _F), lambda i: (0, 0)),
            pl.BlockSpec((_F, _F), lambda i: (0, 0)),
            pl.BlockSpec((_F, 1), lambda i: (0, 0)),
        ],
        out_specs=[
            pl.BlockSpec((_TN, 1), lambda i: (i, 0)),
            pl.BlockSpec((_TN, _F), lambda i: (i, 0)),
        ],
        out_shape=[
            jax.ShapeDtypeStruct((_N, 1), jnp.float32),
            jax.ShapeDtypeStruct((_N, _F), jnp.float32),
        ],
    )(p0, p1, p2, p3, node_feats, spec2, wskip_flat, wprod2, wpl, wread)


def kernel(vectors, node_feats, node_specie, radial_embedding, senders,
           receivers, W_skip, Wr1, br1, Wr2, br2, W_lin, w_prod, W_prodlin,
           W_read):
    ea, eb = _EA, _E - _EA
    snd = senders.astype(jnp.int32)
    rcv = receivers.astype(jnp.int32)
    snd3a = snd[:ea].reshape(_NW, ea // _NW // _CH, _CH)
    snd3b = snd[ea:].reshape(_NW, eb // _NW // _CH, _CH)
    rcv3a = rcv[:ea].reshape(_NW, ea // _NW // _CH, _CH)
    rcv3b = rcv[ea:].reshape(_NW, eb // _NW // _CH, _CH)
    wlin_bf = W_lin.astype(jnp.bfloat16)
    bsel = jnp.repeat(jnp.eye(_SHD, dtype=jnp.bfloat16), _F, axis=1)
    zeros_tile = jnp.zeros((_NPT, _F), jnp.float32)

    gather_a, scatter_a = _sc_impls(ea)
    gather_b, scatter_b = _sc_impls(eb)
    ct = _tc_coef(vectors.T, radial_embedding.T, Wr1.T,
                  br1.reshape(64, 1), Wr2.T, br2.reshape(_SHD, 1))
    h_a = gather_a(node_feats, snd3a)
    h_b = gather_b(node_feats, snd3b)
    m_a = _tc_edge(ct, h_a, bsel, wlin_bf, ea, 0)
    partials_a = scatter_a(m_a, rcv3a, zeros_tile)
    m_b = _tc_edge(ct, h_b, bsel, wlin_bf, eb, ea // _TE)
    partials_b = scatter_b(m_b, rcv3b, zeros_tile)

    spec2 = node_specie.astype(jnp.int32).reshape(_N, 1)
    wskip_flat = W_skip.reshape(_NSPEC * _F, _F)
    wprod2 = w_prod.reshape(_NSPEC, 3 * _F)
    node_outputs, feats = _tc_node(partials_a[0, :_N], partials_a[1, :_N],
                                   partials_b[0, :_N], partials_b[1, :_N],
                                   node_feats, spec2,
                                   wskip_flat, wprod2, W_prodlin, W_read)
    return node_outputs, feats
